# baseline (speedup 1.0000x reference)
import functools
import jax
import jax.numpy as jnp
from jax import lax
from jax.experimental import pallas as pl
from jax.experimental.pallas import tpu as pltpu
from jax.experimental.pallas import tpu_sc as plsc

NC = 2
NS = 16
NW = NC * NS
ROWS_BLK = 512
HI_SHIFT = 6
LO_MASK = 63


@functools.partial(jax.jit, static_argnums=(3, 4))
def _fill(thsc, tlsc, idx_flat, n_idx, d):
  rows_per_w = n_idx // NW
  nb = rows_per_w // ROWS_BLK
  assert rows_per_w % ROWS_BLK == 0 and nb % 2 == 0 and nb >= 6
  w_blk = ROWS_BLK * d
  hd = d // 2

  mesh = plsc.VectorSubcoreMesh(core_axis_name="c", subcore_axis_name="s")

  @functools.partial(
      pl.kernel,
      out_type=jax.ShapeDtypeStruct((n_idx * d,), jnp.float32),
      mesh=mesh,
      compiler_params=pltpu.CompilerParams(
          use_tc_tiling_on_sc=False, needs_layout_passes=False),
      scratch_types=[
          pltpu.VMEM((thsc.shape[0],), jnp.float32),
          pltpu.VMEM((tlsc.shape[0],), jnp.float32),
          pltpu.VMEM((2, ROWS_BLK), jnp.int32),
          pltpu.VMEM((2, w_blk), jnp.float32),
          pltpu.SemaphoreType.DMA,
          pltpu.SemaphoreType.DMA,
          pltpu.SemaphoreType.DMA,
          pltpu.SemaphoreType.DMA,
      ],
  )
  def k(thsc_hbm, tlsc_hbm, idx_hbm, out_hbm, th_v, tl_v, idx_v, rows_v,
        isem0, isem1, ssem0, ssem1):
    isem = (isem0, isem1)
    ssem = (ssem0, ssem1)
    wid = lax.axis_index("s") * NC + lax.axis_index("c")
    base = wid * rows_per_w

    pltpu.sync_copy(thsc_hbm, th_v)
    pltpu.sync_copy(tlsc_hbm, tl_v)

    iota2 = lax.iota(jnp.int32, 16) * 2

    def idx_copy(b, p):
      return pltpu.make_async_copy(
          idx_hbm.at[pl.ds((base + b * ROWS_BLK), ROWS_BLK)],
          idx_v.at[p], isem[p])

    def store_copy(b, p):
      return pltpu.make_async_copy(
          rows_v.at[p],
          out_hbm.at[pl.ds((base + b * ROWS_BLK) * d, w_blk)], ssem[p])

    def fill(p):
      @plsc.parallel_loop(0, ROWS_BLK, 16)
      def _(r0):
        posv = idx_v[p, pl.ds(r0, 16)]
        hiv = jnp.where(posv == 0, thsc.shape[0] // d - 1,
                        lax.shift_right_logical(posv, HI_SHIFT))
        ohv = hiv * d
        olv = (posv & LO_MASK) * d
        for t in range(16):
          oh = ohv[t]
          ol = olv[t]
          sh0 = th_v[pl.ds(oh, 16)]
          sh1 = th_v[pl.ds(oh + 16, 16)]
          ch0 = th_v[pl.ds(oh + hd, 16)]
          ch1 = th_v[pl.ds(oh + hd + 16, 16)]
          sl0 = tl_v[pl.ds(ol, 16)]
          sl1 = tl_v[pl.ds(ol + 16, 16)]
          cl0 = tl_v[pl.ds(ol + hd, 16)]
          cl1 = tl_v[pl.ds(ol + hd + 16, 16)]
          s0 = sh0 * cl0 + ch0 * sl0
          s1 = sh1 * cl1 + ch1 * sl1
          c0 = ch0 * cl0 - sh0 * sl0
          c1 = ch1 * cl1 - sh1 * sl1
          ev = iota2 + (r0 + t) * d
          plsc.store_scatter(rows_v.at[p], [ev], s0)
          plsc.store_scatter(rows_v.at[p], [ev + 1], c0)
          plsc.store_scatter(rows_v.at[p], [ev + hd], s1)
          plsc.store_scatter(rows_v.at[p], [ev + hd + 1], c1)

    idx_copy(0, 0).start()
    idx_copy(1, 1).start()
    idx_copy(0, 0).wait()
    fill(0)
    store_copy(0, 0).start()
    idx_copy(2, 0).start()
    idx_copy(1, 1).wait()
    fill(1)
    store_copy(1, 1).start()
    idx_copy(3, 1).start()

    @pl.loop(1, nb // 2 - 1)
    def _(i):
      b = i * 2
      store_copy(b - 2, 0).wait()
      idx_copy(b, 0).wait()
      fill(0)
      store_copy(b, 0).start()
      idx_copy(b + 2, 0).start()
      store_copy(b - 1, 1).wait()
      idx_copy(b + 1, 1).wait()
      fill(1)
      store_copy(b + 1, 1).start()
      idx_copy(b + 3, 1).start()

    store_copy(nb - 4, 0).wait()
    idx_copy(nb - 2, 0).wait()
    fill(0)
    store_copy(nb - 2, 0).start()
    store_copy(nb - 3, 1).wait()
    idx_copy(nb - 1, 1).wait()
    fill(1)
    store_copy(nb - 1, 1).start()

    store_copy(nb - 2, 0).wait()
    store_copy(nb - 1, 1).wait()

  return k(thsc, tlsc, idx_flat)


def _deinterleave(t):
  return jnp.concatenate([t[:, 0::2], t[:, 1::2]], axis=1)


def kernel(src_pos, table):
  b, h = src_pos.shape
  n, d = table.shape
  unit = jnp.tile(jnp.asarray([0.0, 1.0], table.dtype), d // 2)
  th = table[:: (1 << HI_SHIFT)].at[0].set(unit)
  th = jnp.concatenate([th, jnp.zeros((1, d), table.dtype)], axis=0)
  tl = table[: (1 << HI_SHIFT)].at[0].set(unit)
  thsc = _deinterleave(th).reshape(-1)
  tlsc = _deinterleave(tl).reshape(-1)
  out = _fill(thsc, tlsc, src_pos.reshape(-1), b * h, d)
  return out.reshape(b, h, d)
